# baseline (speedup 1.0000x reference)
import functools
import math

import jax
import jax.numpy as jnp
from jax import lax
from jax.experimental import pallas as pl
from jax.experimental.pallas import tpu as pltpu
from jax.experimental.pallas import tpu_sc as plsc

VOCAB = 1000000
D = 64
B = 4096
L = 200
N = B * L
SCALE = math.sqrt(D)

NUM_CORES = 2
NUM_SUBCORES = 16
NW = NUM_CORES * NUM_SUBCORES
PER_W = N // NW
CHUNK = 512
NCHUNK = PER_W // CHUNK
NBUF = 2

_mesh = plsc.VectorSubcoreMesh(core_axis_name="c", subcore_axis_name="s")


@functools.partial(
    pl.kernel,
    mesh=_mesh,
    compiler_params=pltpu.CompilerParams(use_tc_tiling_on_sc=False),
    out_type=jax.ShapeDtypeStruct((N, D), jnp.float32),
    scratch_types=[
        pltpu.VMEM((NCHUNK, CHUNK), jnp.int32),
        pltpu.VMEM((CHUNK, D), jnp.float32),
        pltpu.VMEM((CHUNK, D), jnp.float32),
        pltpu.SemaphoreType.DMA,
        pltpu.SemaphoreType.DMA,
        pltpu.SemaphoreType.DMA,
        pltpu.SemaphoreType.DMA,
    ],
)
def _embed_sc(x_hbm, table_hbm, out_hbm, idx_v, rows0, rows1,
              gsem0, gsem1, wsem0, wsem1):
    wid = lax.axis_index("s") * NUM_CORES + lax.axis_index("c")
    base = wid * PER_W

    rows = (rows0, rows1)
    gsem = (gsem0, gsem1)
    wsem = (wsem0, wsem1)

    pltpu.sync_copy(x_hbm.at[wid], idx_v)

    def gather_start(b, g):
        pltpu.make_async_copy(table_hbm.at[idx_v.at[g]], rows[b], gsem[b]).start()

    def gather_wait(b):
        pltpu.make_async_copy(table_hbm.at[idx_v.at[0]], rows[b], gsem[b]).wait()

    def write_start(b, g):
        dst = out_hbm.at[pl.ds(base + g * CHUNK, CHUNK)]
        pltpu.make_async_copy(rows[b], dst, wsem[b]).start()

    def write_wait(b):
        dst = out_hbm.at[pl.ds(base, CHUNK)]
        pltpu.make_async_copy(rows[b], dst, wsem[b]).wait()

    def scale_rows(b):
        ref = rows[b]

        def body(r, carry):
            for j in range(D // 16):
                sl = (r, pl.ds(j * 16, 16))
                ref[sl] = ref[sl] * SCALE
            return carry

        lax.fori_loop(0, CHUNK, body, 0, unroll=2)

    gather_start(0, 0)
    gather_start(1, 1)

    def outer(i, carry):
        for b in range(NBUF):
            g = i * NBUF + b
            gather_wait(b)
            scale_rows(b)
            write_start(b, g)
            write_wait(b)

            @pl.when(g + NBUF < NCHUNK)
            def _():
                gather_start(b, g + NBUF)
        return carry

    lax.fori_loop(0, NCHUNK // NBUF, outer, 0)


def kernel(x, table):
    xf = x.reshape(NW, NCHUNK, CHUNK)
    out = _embed_sc(xf, table)
    return out.reshape(B, L, D)


# speedup vs baseline: 1.4980x; 1.4980x over previous
import functools
import math

import jax
import jax.numpy as jnp
from jax import lax
from jax.experimental import pallas as pl
from jax.experimental.pallas import tpu as pltpu
from jax.experimental.pallas import tpu_sc as plsc

VOCAB = 1000000
D = 64
B = 4096
L = 200
N = B * L
SCALE = math.sqrt(D)

NUM_CORES = 2
NUM_SUBCORES = 16
NW = NUM_CORES * NUM_SUBCORES
PER_W = N // NW
CHUNK = 256
NCHUNK = PER_W // CHUNK
NBUF = 2

_mesh = plsc.VectorSubcoreMesh(core_axis_name="c", subcore_axis_name="s")


@functools.partial(
    pl.kernel,
    mesh=_mesh,
    out_type=jax.ShapeDtypeStruct((N, D), jnp.float32),
    scratch_types=[
        pltpu.VMEM((NCHUNK, CHUNK), jnp.int32),
        pltpu.VMEM((CHUNK, D), jnp.float32),
        pltpu.VMEM((CHUNK, D), jnp.float32),
        pltpu.SemaphoreType.DMA,
        pltpu.SemaphoreType.DMA,
        pltpu.SemaphoreType.DMA,
        pltpu.SemaphoreType.DMA,
    ],
)
def _embed_sc(x_hbm, table_hbm, out_hbm, idx_v, rows0, rows1,
              gsem0, gsem1, wsem0, wsem1):
    wid = lax.axis_index("s") * NUM_CORES + lax.axis_index("c")
    base = wid * PER_W

    rows = (rows0, rows1)
    gsem = (gsem0, gsem1)
    wsem = (wsem0, wsem1)

    pltpu.sync_copy(x_hbm.at[wid], idx_v)

    def gather_start(b, g):
        def body(jv, carry):
            vec = idx_v[g, pl.ds(jv * 16, 16)]
            for k in range(16):
                pltpu.make_async_copy(
                    table_hbm.at[vec[k]],
                    rows[b].at[jv * 16 + k],
                    gsem[b]).start()
            return carry

        lax.fori_loop(0, CHUNK // 16, body, 0)

    def gather_wait(b):
        pltpu.make_async_copy(
            table_hbm.at[pl.ds(0, CHUNK)], rows[b], gsem[b]).wait()

    def write_start(b, g):
        dst = out_hbm.at[pl.ds(base + g * CHUNK, CHUNK)]
        pltpu.make_async_copy(rows[b], dst, wsem[b]).start()

    def write_wait(b):
        dst = out_hbm.at[pl.ds(base, CHUNK)]
        pltpu.make_async_copy(rows[b], dst, wsem[b]).wait()

    def scale_rows(b):
        ref = rows[b]

        def body(r, carry):
            for j in range(D // 16):
                sl = (r, pl.ds(j * 16, 16))
                ref[sl] = ref[sl] * SCALE
            return carry

        lax.fori_loop(0, CHUNK, body, 0, unroll=2)

    gather_start(0, 0)
    gather_start(1, 1)

    def outer(i, carry):
        for b in range(NBUF):
            g = i * NBUF + b
            gather_wait(b)
            scale_rows(b)
            write_start(b, g)
            write_wait(b)

            @pl.when(g + NBUF < NCHUNK)
            def _():
                gather_start(b, g + NBUF)
        return carry

    lax.fori_loop(0, NCHUNK // NBUF, outer, 0)


def kernel(x, table):
    xf = x.reshape(NW, NCHUNK, CHUNK)
    out = _embed_sc(xf, table)
    return out.reshape(B, L, D)
